# Initial kernel scaffold; baseline (speedup 1.0000x reference)
import jax
import jax.numpy as jnp
from jax.experimental import pallas as pl
from jax.experimental.pallas import tpu as pltpu

_BN_EPS = 1e-5


def _softsign(z):
    return z / (1.0 + jnp.abs(z))


def _s_block(S):
    for cand in (2048, 1024, 512, 256, 128):
        if S % cand == 0:
            return cand
    return S


def _pool_kernel(x_ref, qp_ref, pooled_ref):
    X = x_ref[0]
    T, S = X.shape
    ones_t = jnp.ones((1, T), jnp.float32)
    ones_s = jnp.ones((1, S), jnp.float32)

    xs = jnp.dot(ones_t, X, preferred_element_type=jnp.float32)
    pa = jax.lax.dot_general(
        ones_s, X, (((1,), (1,)), ((), ())),
        preferred_element_type=jnp.float32) * (1.0 / S)
    pbc = jnp.dot(xs, qp_ref[...], preferred_element_type=jnp.float32)

    pooled_ref[0] = jnp.concatenate([pa, pbc], axis=1)


def _chain_apply_kernel(pooled_ref, wa_ref, ba_ref, wb_ref, bb_ref,
                        wc_ref, bc_ref, w_ref, qt_ref, pt_ref, x_ref,
                        out_ref, scale_scr, spat_scr):
    b = pl.program_id(1)
    Bsz = pooled_ref.shape[0]
    R, T = wa_ref.shape[0], wa_ref.shape[1]
    Nx = wb_ref.shape[1]
    Ny = wc_ref.shape[1]
    D = T + Nx + Ny
    S_blk = x_ref.shape[2]

    @pl.when(b == 0)
    def _chain():
        pooled = pooled_ref[:, 0, :]
        off = jnp.zeros((1, T), jnp.float32)

        for r in range(R):
            pa = pooled[:, 0:T]
            pb = pooled[:, T:T + Nx]
            pc = pooled[:, T + Nx:D]
            av = _softsign(jax.lax.dot_general(
                pa, wa_ref[r], (((1,), (1,)), ((), ())),
                preferred_element_type=jnp.float32) + ba_ref[r])
            bv = _softsign(jax.lax.dot_general(
                pb, wb_ref[r], (((1,), (1,)), ((), ())),
                preferred_element_type=jnp.float32) + bb_ref[r])
            cv = _softsign(jax.lax.dot_general(
                pc, wc_ref[r], (((1,), (1,)), ((), ())),
                preferred_element_type=jnp.float32) + bc_ref[r])

            bbar = jnp.mean(bv, axis=1, keepdims=True)
            cbar = jnp.mean(cv, axis=1, keepdims=True)
            b2 = jnp.mean(bv * bv, axis=1, keepdims=True)
            c2 = jnp.mean(cv * cv, axis=1, keepdims=True)
            mu = jnp.mean(av * (bbar * cbar), axis=0, keepdims=True)
            m2 = jnp.mean((av * av) * (b2 * c2), axis=0, keepdims=True)
            var = jnp.maximum(m2 - mu * mu, 0.0)
            inv = jax.lax.rsqrt(var + _BN_EPS)

            wr = w_ref[r]
            scale_scr[:, r, :] = (wr * inv) * av
            spat_scr[:, r, :] = (
                jnp.dot(bv, qt_ref[...], preferred_element_type=jnp.float32) *
                jnp.dot(cv, pt_ref[...], preferred_element_type=jnp.float32))
            off = off + wr * (inv * mu)

            if r + 1 < R:
                pa_n = pa - inv * (av * (bbar * cbar) - mu)
                a1 = jnp.mean(inv * av, axis=1, keepdims=True)
                m1 = jnp.mean(inv * mu, axis=1, keepdims=True)
                pb_n = pb - (bv * (cbar * a1) - m1)
                pc_n = pc - (cv * (bbar * a1) - m1)
                pooled = jnp.concatenate([pa_n, pb_n, pc_n], axis=1)

        scale_scr[:, R, :] = jnp.broadcast_to(-off, (Bsz, T))
        spat_scr[:, R, :] = jnp.ones((Bsz, S_blk), jnp.float32)

    sc = scale_scr[b]
    sp = spat_scr[b]
    delta = jax.lax.dot_general(
        sc, sp, (((0,), (0,)), ((), ())),
        preferred_element_type=jnp.float32)
    out_ref[0] = x_ref[0] + delta


def _cp_forward(x, Wa, ba, Wb, bb, Wc, bc, w):
    B, T, Nx, Ny = x.shape
    S = Nx * Ny
    R = Wa.shape[0]
    R1 = R + 1
    D = T + Nx + Ny

    x2 = x.reshape(B, T, S)

    s_idx = jnp.arange(S, dtype=jnp.int32)
    Q = (s_idx[:, None] // Ny == jnp.arange(Nx, dtype=jnp.int32)[None, :]
         ).astype(jnp.float32)
    P = (s_idx[:, None] % Ny == jnp.arange(Ny, dtype=jnp.int32)[None, :]
         ).astype(jnp.float32)
    QP = jnp.concatenate([Q * (1.0 / (T * Ny)), P * (1.0 / (T * Nx))], axis=1)

    pooled = pl.pallas_call(
        _pool_kernel,
        out_shape=jax.ShapeDtypeStruct((B, 1, D), jnp.float32),
        grid=(B,),
        in_specs=[
            pl.BlockSpec((1, T, S), lambda b: (b, 0, 0)),
            pl.BlockSpec((S, Nx + Ny), lambda b: (0, 0)),
        ],
        out_specs=pl.BlockSpec((1, 1, D), lambda b: (b, 0, 0)),
        compiler_params=pltpu.CompilerParams(
            dimension_semantics=("core_parallel",)),
    )(x2, QP)

    S_blk = _s_block(S)
    NS = S // S_blk
    smem = pl.BlockSpec(memory_space=pltpu.MemorySpace.SMEM)

    out2 = pl.pallas_call(
        _chain_apply_kernel,
        out_shape=jax.ShapeDtypeStruct((B, T, S), x.dtype),
        grid=(NS, B),
        in_specs=[
            pl.BlockSpec((B, 1, D), lambda s, b: (0, 0, 0)),
            pl.BlockSpec((R, T, T), lambda s, b: (0, 0, 0)),
            pl.BlockSpec((R, 1, T), lambda s, b: (0, 0, 0)),
            pl.BlockSpec((R, Nx, Nx), lambda s, b: (0, 0, 0)),
            pl.BlockSpec((R, 1, Nx), lambda s, b: (0, 0, 0)),
            pl.BlockSpec((R, Ny, Ny), lambda s, b: (0, 0, 0)),
            pl.BlockSpec((R, 1, Ny), lambda s, b: (0, 0, 0)),
            smem,
            pl.BlockSpec((Nx, S_blk), lambda s, b: (0, s)),
            pl.BlockSpec((Ny, S_blk), lambda s, b: (0, s)),
            pl.BlockSpec((1, T, S_blk), lambda s, b: (b, 0, s)),
        ],
        out_specs=pl.BlockSpec((1, T, S_blk), lambda s, b: (b, 0, s)),
        scratch_shapes=[
            pltpu.VMEM((B, R1, T), jnp.float32),
            pltpu.VMEM((B, R1, S_blk), jnp.float32),
        ],
        compiler_params=pltpu.CompilerParams(
            dimension_semantics=("core_parallel", "arbitrary")),
    )(pooled, Wa, ba, Wb, bb, Wc, bc, w, Q.T, P.T, x2)

    return out2.reshape(B, T, Nx, Ny)


def kernel(x, Wa, ba, Wb, bb, Wc, bc, w):
    return _cp_forward(x, Wa, ba, Wb, bb, Wc, bc, w)


# speedup vs baseline: 1.0496x; 1.0496x over previous
import jax
import jax.numpy as jnp
from jax.experimental import pallas as pl
from jax.experimental.pallas import tpu as pltpu

_BN_EPS = 1e-5


def _softsign(z):
    return z / (1.0 + jnp.abs(z))


def _s_block(S):
    for cand in (2048, 1024, 512, 256, 128):
        if S % cand == 0:
            return cand
    return S


def _pool_kernel(x_ref, qp_ref, pooled_ref):
    X = x_ref[0]
    T, S = X.shape
    ones_t = jnp.ones((1, T), jnp.float32)
    ones_s = jnp.ones((1, S), jnp.float32)

    xs = jnp.dot(ones_t, X, preferred_element_type=jnp.float32)
    pa = jax.lax.dot_general(
        ones_s, X, (((1,), (1,)), ((), ())),
        preferred_element_type=jnp.float32) * (1.0 / S)
    pbc = jnp.dot(xs, qp_ref[...], preferred_element_type=jnp.float32)

    pooled_ref[0] = jnp.concatenate([pa, pbc], axis=1)


def _chain_apply_kernel(pooled_ref, wa_ref, ba_ref, wb_ref, bb_ref,
                        wc_ref, bc_ref, w_ref, qt_ref, pt_ref, x_ref,
                        out_ref, scale_scr, spat_scr):
    b = pl.program_id(1)
    Bsz = pooled_ref.shape[0]
    R, T = wa_ref.shape[0], wa_ref.shape[1]
    Nx = wb_ref.shape[1]
    Ny = wc_ref.shape[1]
    D = T + Nx + Ny
    S_blk = x_ref.shape[2]

    @pl.when(b == 0)
    def _chain():
        pooled = pooled_ref[:, 0, :]
        off = jnp.zeros((1, T), jnp.float32)

        for r in range(R):
            pa = pooled[:, 0:T]
            pb = pooled[:, T:T + Nx]
            pc = pooled[:, T + Nx:D]
            av = _softsign(jax.lax.dot_general(
                pa, wa_ref[r], (((1,), (1,)), ((), ())),
                preferred_element_type=jnp.float32) + ba_ref[r])
            bv = _softsign(jax.lax.dot_general(
                pb, wb_ref[r], (((1,), (1,)), ((), ())),
                preferred_element_type=jnp.float32) + bb_ref[r])
            cv = _softsign(jax.lax.dot_general(
                pc, wc_ref[r], (((1,), (1,)), ((), ())),
                preferred_element_type=jnp.float32) + bc_ref[r])

            bbar = jnp.mean(bv, axis=1, keepdims=True)
            cbar = jnp.mean(cv, axis=1, keepdims=True)
            b2 = jnp.mean(bv * bv, axis=1, keepdims=True)
            c2 = jnp.mean(cv * cv, axis=1, keepdims=True)
            mu = jnp.mean(av * (bbar * cbar), axis=0, keepdims=True)
            m2 = jnp.mean((av * av) * (b2 * c2), axis=0, keepdims=True)
            var = jnp.maximum(m2 - mu * mu, 0.0)
            inv = jax.lax.rsqrt(var + _BN_EPS)

            wr = w_ref[r]
            scale_scr[:, r, :] = (wr * inv) * av
            spat_scr[:, r, :] = (
                jnp.dot(bv, qt_ref[...], preferred_element_type=jnp.float32) *
                jnp.dot(cv, pt_ref[...], preferred_element_type=jnp.float32))
            off = off + wr * (inv * mu)

            if r + 1 < R:
                pa_n = pa - inv * (av * (bbar * cbar) - mu)
                a1 = jnp.mean(inv * av, axis=1, keepdims=True)
                m1 = jnp.mean(inv * mu, axis=1, keepdims=True)
                pb_n = pb - (bv * (cbar * a1) - m1)
                pc_n = pc - (cv * (bbar * a1) - m1)
                pooled = jnp.concatenate([pa_n, pb_n, pc_n], axis=1)

        scale_scr[:, R, :] = jnp.broadcast_to(-off, (Bsz, T))
        spat_scr[:, R, :] = jnp.ones((Bsz, S_blk), jnp.float32)

    sc = scale_scr[b]
    sp = spat_scr[b]
    delta = jax.lax.dot_general(
        sc, sp, (((0,), (0,)), ((), ())),
        preferred_element_type=jnp.float32)
    out_ref[0] = x_ref[0] + delta


def _cp_forward(x, Wa, ba, Wb, bb, Wc, bc, w):
    B, T, Nx, Ny = x.shape
    S = Nx * Ny
    R = Wa.shape[0]
    R1 = R + 1
    D = T + Nx + Ny

    x2 = x.reshape(B, T, S)

    s_idx = jnp.arange(S, dtype=jnp.int32)
    Q = (s_idx[:, None] // Ny == jnp.arange(Nx, dtype=jnp.int32)[None, :]
         ).astype(jnp.float32)
    P = (s_idx[:, None] % Ny == jnp.arange(Ny, dtype=jnp.int32)[None, :]
         ).astype(jnp.float32)
    QP = jnp.concatenate([Q * (1.0 / (T * Ny)), P * (1.0 / (T * Nx))], axis=1)

    pooled = pl.pallas_call(
        _pool_kernel,
        out_shape=jax.ShapeDtypeStruct((B, 1, D), jnp.float32),
        grid=(B,),
        in_specs=[
            pl.BlockSpec((1, T, S), lambda b: (b, 0, 0)),
            pl.BlockSpec((S, Nx + Ny), lambda b: (0, 0)),
        ],
        out_specs=pl.BlockSpec((1, 1, D), lambda b: (b, 0, 0)),
        compiler_params=pltpu.CompilerParams(
            dimension_semantics=("arbitrary",)),
    )(x2, QP)

    S_blk = _s_block(S)
    NS = S // S_blk
    smem = pl.BlockSpec(memory_space=pltpu.MemorySpace.SMEM)

    out2 = pl.pallas_call(
        _chain_apply_kernel,
        out_shape=jax.ShapeDtypeStruct((B, T, S), x.dtype),
        grid=(NS, B),
        in_specs=[
            pl.BlockSpec((B, 1, D), lambda s, b: (0, 0, 0)),
            pl.BlockSpec((R, T, T), lambda s, b: (0, 0, 0)),
            pl.BlockSpec((R, 1, T), lambda s, b: (0, 0, 0)),
            pl.BlockSpec((R, Nx, Nx), lambda s, b: (0, 0, 0)),
            pl.BlockSpec((R, 1, Nx), lambda s, b: (0, 0, 0)),
            pl.BlockSpec((R, Ny, Ny), lambda s, b: (0, 0, 0)),
            pl.BlockSpec((R, 1, Ny), lambda s, b: (0, 0, 0)),
            smem,
            pl.BlockSpec((Nx, S_blk), lambda s, b: (0, s)),
            pl.BlockSpec((Ny, S_blk), lambda s, b: (0, s)),
            pl.BlockSpec((1, T, S_blk), lambda s, b: (b, 0, s)),
        ],
        out_specs=pl.BlockSpec((1, T, S_blk), lambda s, b: (b, 0, s)),
        scratch_shapes=[
            pltpu.VMEM((B, R1, T), jnp.float32),
            pltpu.VMEM((B, R1, S_blk), jnp.float32),
        ],
        compiler_params=pltpu.CompilerParams(
            dimension_semantics=("arbitrary", "arbitrary")),
    )(pooled, Wa, ba, Wb, bb, Wc, bc, w, Q.T, P.T, x2)

    return out2.reshape(B, T, Nx, Ny)


def kernel(x, Wa, ba, Wb, bb, Wc, bc, w):
    return _cp_forward(x, Wa, ba, Wb, bb, Wc, bc, w)
